# baseline (speedup 1.0000x reference)
import functools

import jax
import jax.numpy as jnp
import numpy as np
from jax.experimental import pallas as pl
from jax.experimental.pallas import tpu as pltpu


def _lerp_weights(n_in: int, n_out: int) -> np.ndarray:
    if n_in == 1:
        return np.ones((n_out, 1), dtype=np.float32)
    pos = np.arange(n_out, dtype=np.float64) * (n_in - 1) / (n_out - 1)
    lo = np.minimum(pos.astype(np.int64), n_in - 2)
    frac = (pos - lo).astype(np.float32)
    w = np.zeros((n_out, n_in), dtype=np.float32)
    rows = np.arange(n_out)
    w[rows, lo] = 1.0 - frac
    w[rows, lo + 1] = frac
    return w


def _tile_body(x_ref, ah_ref, awt_ref, o_ref):
    t, h, w = x_ref.shape
    w2 = awt_ref.shape[1]

    xb = x_ref[...].astype(jnp.bfloat16).reshape(t * h, w)
    tmp = jnp.dot(xb, awt_ref[...], preferred_element_type=jnp.float32)
    tmpb = tmp.astype(jnp.bfloat16).reshape(t, h, w2)

    ah = ah_ref[...]
    for i in range(t):
        o_ref[i] = jnp.dot(ah, tmpb[i], preferred_element_type=jnp.float32)


@jax.jit
def _upsample2x(x: jnp.ndarray) -> jnp.ndarray:
    B, C, H, W = x.shape
    H2, W2 = 2 * H, 2 * W
    N = B * C

    a_h = jnp.asarray(_lerp_weights(H, H2), dtype=jnp.bfloat16)
    a_w_t = jnp.asarray(_lerp_weights(W, W2).T, dtype=jnp.bfloat16)

    T = 64
    while N % T:
        T //= 2
    x_stacked = x.reshape(N, H, W)

    out = pl.pallas_call(
        _tile_body,
        out_shape=jax.ShapeDtypeStruct((N, H2, W2), x.dtype),
        grid=(N // T,),
        in_specs=[
            pl.BlockSpec((T, H, W), lambda i: (i, 0, 0)),
            pl.BlockSpec((H2, H), lambda i: (0, 0)),
            pl.BlockSpec((W, W2), lambda i: (0, 0)),
        ],
        out_specs=pl.BlockSpec((T, H2, W2), lambda i: (i, 0, 0)),
        compiler_params=pltpu.CompilerParams(
            dimension_semantics=("parallel",),
            vmem_limit_bytes=64 * 1024 * 1024,
        ),
    )(x_stacked, a_h, a_w_t)

    return out.reshape(B, C, H2, W2)


def kernel(x):
    return _upsample2x(x)
